# baseline (speedup 1.0000x reference)
import functools

import jax
import jax.numpy as jnp
from jax.experimental import pallas as pl
from jax.experimental.pallas import tpu as pltpu

N = 8192
R = 65
ONEHOT = 512
GLOB = 32
FEAT = 256
HID = 32
NH = 16
TFS = 64
C13 = 13

BLOCK = 64


def _body(tfil_ref, tmask_ref, tedge_ref, tg_ref, toh_ref, tot_ref,
          W2_ref, Wse_ref, bse_ref, Wagg_ref, Woh1_ref, Woh2_ref, out_ref):
    toh = toh_ref[...]
    M = jnp.dot(toh, W2_ref[...],
                preferred_element_type=jnp.float32)
    B = toh.shape[0]
    emb = jnp.zeros((B, R, HID), jnp.float32)
    for c in range(C13):
        A_c = tot_ref[c]
        M_c = M[:, c * HID:(c + 1) * HID]
        emb = emb + A_c[:, :, None] * M_c[:, None, :]
    emb = jnp.maximum(emb, 0.0)
    emb2 = jax.lax.dot_general(
        emb, Wse_ref[...], (((2,), (0,)), ((), ())),
        preferred_element_type=jnp.float32) + bse_ref[...][None, :, :]

    tfil = tfil_ref[...]
    tmask = tmask_ref[...]
    tedge = tedge_ref[...]
    tfa = -jnp.log(jnp.clip(1.0 - tfil * tmask, 1e-9, None))
    tfb = -jnp.log(jnp.clip(1.0 - (1.0 - tfil) * tmask, 1e-9, None))

    big = jnp.float32(1e38)
    em = (tedge > 0.0).astype(jnp.float32)
    scale = jnp.concatenate([
        jnp.broadcast_to(tfa[:, :, None], tfa.shape + (NH,)),
        jnp.broadcast_to(tfb[:, :, None], tfb.shape + (NH,)),
        jnp.broadcast_to(tedge[:, :, None], tedge.shape + (NH,)),
    ], axis=2)
    moff = jnp.concatenate([
        jnp.broadcast_to(((tmask - 1.0) * big)[:, :, None], tmask.shape + (2 * NH,)),
        jnp.broadcast_to(((em - 1.0) * big)[:, :, None], em.shape + (NH,)),
    ], axis=2)
    mx = jnp.max(emb2 * scale + moff, axis=1)
    m_any = jnp.max(tmask, axis=1)[:, None]
    e_any = jnp.max(em, axis=1)[:, None]
    anyv = jnp.concatenate([
        jnp.broadcast_to(m_any, (B, 2 * NH)),
        jnp.broadcast_to(e_any, (B, NH))], axis=1)
    agg = mx * anyv
    out = jnp.dot(agg, Wagg_ref[...], preferred_element_type=jnp.float32)
    out = out + jnp.dot(toh, Woh1_ref[...], preferred_element_type=jnp.float32)
    out = out + jnp.dot(tg_ref[...], Woh2_ref[...],
                        preferred_element_type=jnp.float32)
    out_ref[...] = out


@jax.jit
def kernel(table_filter, table_mask, table_edge, table_global, table_onehot,
           table_others, W_onehot_emb, W_schema_prep, W_se, b_se,
           W_onehot_transform, W_edge_fc, W_left, W_right, W_tail):
    n = table_onehot.shape[0]
    W_emb3 = W_onehot_emb.reshape(ONEHOT, HID, HID)
    W2 = jnp.einsum('ch,shk->sck', W_schema_prep, W_emb3).reshape(
        ONEHOT, C13 * HID)
    Wagg = jnp.concatenate([
        W_left @ W_tail[HID:2 * HID],
        W_right @ W_tail[2 * HID:3 * HID],
        W_edge_fc @ W_tail[0:HID]], axis=0)
    Woh = W_onehot_transform @ W_tail[3 * HID:]
    Woh1 = Woh[:ONEHOT]
    Woh2 = Woh[ONEHOT:]
    bse2 = b_se.reshape(1, 3 * NH)

    tot_t = jnp.transpose(table_others, (2, 0, 1))

    grid = n // BLOCK
    out = pl.pallas_call(
        _body,
        grid=(grid,),
        in_specs=[
            pl.BlockSpec((BLOCK, R), lambda i: (i, 0)),
            pl.BlockSpec((BLOCK, R), lambda i: (i, 0)),
            pl.BlockSpec((BLOCK, R), lambda i: (i, 0)),
            pl.BlockSpec((BLOCK, GLOB), lambda i: (i, 0)),
            pl.BlockSpec((BLOCK, ONEHOT), lambda i: (i, 0)),
            pl.BlockSpec((C13, BLOCK, R), lambda i: (0, i, 0)),
            pl.BlockSpec((ONEHOT, C13 * HID), lambda i: (0, 0)),
            pl.BlockSpec((HID, 3 * NH), lambda i: (0, 0)),
            pl.BlockSpec((1, 3 * NH), lambda i: (0, 0)),
            pl.BlockSpec((3 * NH, FEAT), lambda i: (0, 0)),
            pl.BlockSpec((ONEHOT, FEAT), lambda i: (0, 0)),
            pl.BlockSpec((GLOB, FEAT), lambda i: (0, 0)),
        ],
        out_specs=pl.BlockSpec((BLOCK, FEAT), lambda i: (i, 0)),
        out_shape=jax.ShapeDtypeStruct((n, FEAT), jnp.float32),
    )(table_filter, table_mask, table_edge, table_global, table_onehot,
      tot_t, W2, W_se, bse2, Wagg, Woh1, Woh2)
    return out


# speedup vs baseline: 4.3190x; 4.3190x over previous
import functools

import jax
import jax.numpy as jnp
from jax.experimental import pallas as pl
from jax.experimental.pallas import tpu as pltpu

N = 8192
R = 65
ONEHOT = 512
GLOB = 32
FEAT = 256
HID = 32
NH = 16
TFS = 64
C13 = 13

BLOCK = 64


def _body(aux_ref, tmask_ref, tedge_ref, tg_ref, toh_ref, tot_ref,
          W2_ref, Wse_ref, sel_ref, Wagg_ref, Woh1_ref, Woh2_ref, out_ref):
    toh = toh_ref[...]
    M = jnp.dot(toh, W2_ref[...],
                preferred_element_type=jnp.float32)
    B = toh.shape[0]
    M3 = M.reshape(B, C13, HID)
    emb = jax.lax.dot_general(
        tot_ref[...], M3, (((2,), (1,)), ((0,), (0,))),
        preferred_element_type=jnp.float32)
    emb = jnp.maximum(emb, 0.0)
    emb2 = jax.lax.dot_general(
        emb, Wse_ref[...], (((2,), (1,)), ((0,), (0,))),
        preferred_element_type=jnp.float32)

    sm = jax.lax.dot_general(
        aux_ref[...], sel_ref[...], (((2,), (1,)), ((0,), (0,))),
        preferred_element_type=jnp.float32)
    mx = jnp.max(emb2 * sm[:, :, :48] + sm[:, :, 48:], axis=1)
    tmask = tmask_ref[...]
    em = (tedge_ref[...] > 0.0).astype(jnp.float32)
    m_any = jnp.max(tmask, axis=1)[:, None]
    e_any = jnp.max(em, axis=1)[:, None]
    anyv = jnp.concatenate([
        jnp.broadcast_to(m_any, (B, 2 * NH)),
        jnp.broadcast_to(e_any, (B, NH))], axis=1)
    agg = mx * anyv
    out = jnp.dot(agg, Wagg_ref[...], preferred_element_type=jnp.float32)
    out = out + jnp.dot(toh, Woh1_ref[...], preferred_element_type=jnp.float32)
    out = out + jnp.dot(tg_ref[...], Woh2_ref[...],
                        preferred_element_type=jnp.float32)
    out_ref[...] = out


@jax.jit
def kernel(table_filter, table_mask, table_edge, table_global, table_onehot,
           table_others, W_onehot_emb, W_schema_prep, W_se, b_se,
           W_onehot_transform, W_edge_fc, W_left, W_right, W_tail):
    n = table_onehot.shape[0]
    W_emb3 = W_onehot_emb.reshape(ONEHOT, HID, HID)
    W2 = jnp.einsum('ch,shk->sck', W_schema_prep, W_emb3).reshape(
        ONEHOT, C13 * HID)
    Wagg = jnp.concatenate([
        W_left @ W_tail[HID:2 * HID],
        W_right @ W_tail[2 * HID:3 * HID],
        W_edge_fc @ W_tail[0:HID]], axis=0)
    Woh = W_onehot_transform @ W_tail[3 * HID:]
    Woh1 = Woh[:ONEHOT]
    Woh2 = Woh[ONEHOT:]
    ID3 = jnp.repeat(jnp.eye(3, dtype=jnp.float32), NH, axis=1)
    Z3 = jnp.zeros((3, 3 * NH), jnp.float32)
    S1 = jnp.concatenate([ID3, Z3], axis=0)
    S2 = jnp.concatenate([Z3, ID3], axis=0)
    sel = jnp.concatenate([S1, S2 + S1 * b_se[None, :]], axis=1)
    selB = jnp.broadcast_to(sel[None], (BLOCK, 6, 96))
    WseB = jnp.broadcast_to(W_se[None], (BLOCK, HID, 3 * NH))

    big = jnp.float32(1e38)
    tfa = -jnp.log(jnp.clip(1.0 - table_filter * table_mask, 1e-9, None))
    tfb = -jnp.log(jnp.clip(1.0 - (1.0 - table_filter) * table_mask, 1e-9, None))
    emn = (table_edge > 0.0).astype(jnp.float32)
    mb = (table_mask - 1.0) * big
    aux = jnp.stack([tfa, tfb, table_edge, mb, mb, (emn - 1.0) * big],
                    axis=-1)

    grid = n // BLOCK
    out = pl.pallas_call(
        _body,
        grid=(grid,),
        in_specs=[
            pl.BlockSpec((BLOCK, R, 6), lambda i: (i, 0, 0)),
            pl.BlockSpec((BLOCK, R), lambda i: (i, 0)),
            pl.BlockSpec((BLOCK, R), lambda i: (i, 0)),
            pl.BlockSpec((BLOCK, GLOB), lambda i: (i, 0)),
            pl.BlockSpec((BLOCK, ONEHOT), lambda i: (i, 0)),
            pl.BlockSpec((BLOCK, R, C13), lambda i: (i, 0, 0)),
            pl.BlockSpec((ONEHOT, C13 * HID), lambda i: (0, 0)),
            pl.BlockSpec((BLOCK, HID, 3 * NH), lambda i: (0, 0, 0)),
            pl.BlockSpec((BLOCK, 6, 96), lambda i: (0, 0, 0)),
            pl.BlockSpec((3 * NH, FEAT), lambda i: (0, 0)),
            pl.BlockSpec((ONEHOT, FEAT), lambda i: (0, 0)),
            pl.BlockSpec((GLOB, FEAT), lambda i: (0, 0)),
        ],
        out_specs=pl.BlockSpec((BLOCK, FEAT), lambda i: (i, 0)),
        out_shape=jax.ShapeDtypeStruct((n, FEAT), jnp.float32),
    )(aux, table_mask, table_edge, table_global, table_onehot,
      table_others, W2, WseB, selB, Wagg, Woh1, Woh2)
    return out


# speedup vs baseline: 4.6149x; 1.0685x over previous
import functools

import jax
import jax.numpy as jnp
from jax.experimental import pallas as pl
from jax.experimental.pallas import tpu as pltpu

N = 8192
R = 65
ONEHOT = 512
GLOB = 32
FEAT = 256
HID = 32
NH = 16
TFS = 64
C13 = 13

BLOCK = 128


def _body(aux_ref, tmask_ref, tedge_ref, tg_ref, toh_ref, tot_ref,
          W2_ref, Wse_ref, sel_ref, Wagg_ref, Woh1_ref, Woh2_ref, out_ref):
    toh = toh_ref[...]
    M = jnp.dot(toh, W2_ref[...],
                preferred_element_type=jnp.float32)
    B = toh.shape[0]
    M3 = M.reshape(B, C13, HID)
    emb = jax.lax.dot_general(
        tot_ref[...], M3, (((2,), (1,)), ((0,), (0,))),
        preferred_element_type=jnp.float32)
    emb = jnp.maximum(emb, 0.0)
    emb2 = jax.lax.dot_general(
        emb, Wse_ref[...], (((2,), (1,)), ((0,), (0,))),
        preferred_element_type=jnp.float32)

    sm = jax.lax.dot_general(
        aux_ref[...], sel_ref[...], (((2,), (1,)), ((0,), (0,))),
        preferred_element_type=jnp.float32)
    mx = jnp.max(emb2 * sm[:, :, :48] + sm[:, :, 48:], axis=1)
    tmask = tmask_ref[...]
    em = (tedge_ref[...] > 0.0).astype(jnp.float32)
    m_any = jnp.max(tmask, axis=1)[:, None]
    e_any = jnp.max(em, axis=1)[:, None]
    anyv = jnp.concatenate([
        jnp.broadcast_to(m_any, (B, 2 * NH)),
        jnp.broadcast_to(e_any, (B, NH))], axis=1)
    agg = mx * anyv
    out = jnp.dot(agg, Wagg_ref[...], preferred_element_type=jnp.float32)
    out = out + jnp.dot(toh, Woh1_ref[...], preferred_element_type=jnp.float32)
    out = out + jnp.dot(tg_ref[...], Woh2_ref[...],
                        preferred_element_type=jnp.float32)
    out_ref[...] = out


@jax.jit
def kernel(table_filter, table_mask, table_edge, table_global, table_onehot,
           table_others, W_onehot_emb, W_schema_prep, W_se, b_se,
           W_onehot_transform, W_edge_fc, W_left, W_right, W_tail):
    n = table_onehot.shape[0]
    W_emb3 = W_onehot_emb.reshape(ONEHOT, HID, HID)
    W2 = jnp.einsum('ch,shk->sck', W_schema_prep, W_emb3).reshape(
        ONEHOT, C13 * HID)
    Wagg = jnp.concatenate([
        W_left @ W_tail[HID:2 * HID],
        W_right @ W_tail[2 * HID:3 * HID],
        W_edge_fc @ W_tail[0:HID]], axis=0)
    Woh = W_onehot_transform @ W_tail[3 * HID:]
    Woh1 = Woh[:ONEHOT]
    Woh2 = Woh[ONEHOT:]
    ID3 = jnp.repeat(jnp.eye(3, dtype=jnp.float32), NH, axis=1)
    Z3 = jnp.zeros((3, 3 * NH), jnp.float32)
    S1 = jnp.concatenate([ID3, Z3], axis=0)
    S2 = jnp.concatenate([Z3, ID3], axis=0)
    sel = jnp.concatenate([S1, S2 + S1 * b_se[None, :]], axis=1)
    selB = jnp.broadcast_to(sel[None], (BLOCK, 6, 96))
    WseB = jnp.broadcast_to(W_se[None], (BLOCK, HID, 3 * NH))

    big = jnp.float32(1e38)
    tfa = -jnp.log(jnp.clip(1.0 - table_filter * table_mask, 1e-9, None))
    tfb = -jnp.log(jnp.clip(1.0 - (1.0 - table_filter) * table_mask, 1e-9, None))
    emn = (table_edge > 0.0).astype(jnp.float32)
    mb = (table_mask - 1.0) * big
    aux = jnp.stack([tfa, tfb, table_edge, mb, mb, (emn - 1.0) * big],
                    axis=-1)

    grid = n // BLOCK
    out = pl.pallas_call(
        _body,
        grid=(grid,),
        in_specs=[
            pl.BlockSpec((BLOCK, R, 6), lambda i: (i, 0, 0)),
            pl.BlockSpec((BLOCK, R), lambda i: (i, 0)),
            pl.BlockSpec((BLOCK, R), lambda i: (i, 0)),
            pl.BlockSpec((BLOCK, GLOB), lambda i: (i, 0)),
            pl.BlockSpec((BLOCK, ONEHOT), lambda i: (i, 0)),
            pl.BlockSpec((BLOCK, R, C13), lambda i: (i, 0, 0)),
            pl.BlockSpec((ONEHOT, C13 * HID), lambda i: (0, 0)),
            pl.BlockSpec((BLOCK, HID, 3 * NH), lambda i: (0, 0, 0)),
            pl.BlockSpec((BLOCK, 6, 96), lambda i: (0, 0, 0)),
            pl.BlockSpec((3 * NH, FEAT), lambda i: (0, 0)),
            pl.BlockSpec((ONEHOT, FEAT), lambda i: (0, 0)),
            pl.BlockSpec((GLOB, FEAT), lambda i: (0, 0)),
        ],
        out_specs=pl.BlockSpec((BLOCK, FEAT), lambda i: (i, 0)),
        out_shape=jax.ShapeDtypeStruct((n, FEAT), jnp.float32),
    )(aux, table_mask, table_edge, table_global, table_onehot,
      table_others, W2, WseB, selB, Wagg, Woh1, Woh2)
    return out


# speedup vs baseline: 5.7443x; 1.2447x over previous
import functools

import jax
import jax.numpy as jnp
from jax.experimental import pallas as pl
from jax.experimental.pallas import tpu as pltpu

N = 8192
R = 65
ONEHOT = 512
GLOB = 32
FEAT = 256
HID = 32
NH = 16
TFS = 64
C13 = 13

BLOCK = 128


def _body(aux_ref, tg_ref, toh_ref, tot_ref,
          W2_ref, Wse_ref, sel_ref, Wagg_ref, Woh1_ref, Woh2_ref, out_ref):
    toh = toh_ref[...]
    M = jnp.dot(toh, W2_ref[...],
                preferred_element_type=jnp.float32)
    B = toh.shape[0]
    M3 = M.reshape(B, HID, C13)
    emb = jax.lax.dot_general(
        tot_ref[...], M3, (((2,), (2,)), ((0,), (0,))),
        preferred_element_type=jnp.float32)
    emb = jnp.maximum(emb, 0.0)
    emb2 = jax.lax.dot_general(
        emb, Wse_ref[...], (((2,), (1,)), ((0,), (0,))),
        preferred_element_type=jnp.float32)

    aux2 = aux_ref[...]
    aux3 = aux2.reshape(B, 6, R)
    sm = jax.lax.dot_general(
        aux3, sel_ref[...], (((1,), (1,)), ((0,), (0,))),
        preferred_element_type=jnp.float32)
    mx = jnp.max(emb2 * sm[:, :, :48] + sm[:, :, 48:], axis=1)
    inv_big = jnp.float32(1e-38)
    m_any = 1.0 + jnp.max(aux2[:, 3 * R:4 * R], axis=1)[:, None] * inv_big
    e_any = 1.0 + jnp.max(aux2[:, 5 * R:6 * R], axis=1)[:, None] * inv_big
    anyv = jnp.concatenate([
        jnp.broadcast_to(m_any, (B, 2 * NH)),
        jnp.broadcast_to(e_any, (B, NH))], axis=1)
    agg = mx * anyv
    out = jnp.dot(agg, Wagg_ref[...], preferred_element_type=jnp.float32)
    out = out + jnp.dot(toh, Woh1_ref[...], preferred_element_type=jnp.float32)
    out = out + jnp.dot(tg_ref[...], Woh2_ref[...],
                        preferred_element_type=jnp.float32)
    out_ref[...] = out


@jax.jit
def kernel(table_filter, table_mask, table_edge, table_global, table_onehot,
           table_others, W_onehot_emb, W_schema_prep, W_se, b_se,
           W_onehot_transform, W_edge_fc, W_left, W_right, W_tail):
    n = table_onehot.shape[0]
    W_emb3 = W_onehot_emb.reshape(ONEHOT, HID, HID)
    W2 = jax.lax.dot_general(
        W_emb3, W_schema_prep,
        (((1,), (1,)), ((), ()))).reshape(ONEHOT, HID * C13)
    Wagg = jnp.concatenate([
        W_left @ W_tail[HID:2 * HID],
        W_right @ W_tail[2 * HID:3 * HID],
        W_edge_fc @ W_tail[0:HID]], axis=0)
    Woh = W_onehot_transform @ W_tail[3 * HID:]
    Woh1 = Woh[:ONEHOT]
    Woh2 = Woh[ONEHOT:]
    ID3 = jnp.repeat(jnp.eye(3, dtype=jnp.float32), NH, axis=1)
    Z3 = jnp.zeros((3, 3 * NH), jnp.float32)
    S1 = jnp.concatenate([ID3, Z3], axis=0)
    S2 = jnp.concatenate([Z3, ID3], axis=0)
    sel = jnp.concatenate([S1, S2 + S1 * b_se[None, :]], axis=1)
    selB = jnp.broadcast_to(sel[None], (BLOCK, 6, 96))
    WseB = jnp.broadcast_to(W_se[None], (BLOCK, HID, 3 * NH))

    big = jnp.float32(1e38)
    tfa = -jnp.log(jnp.clip(1.0 - table_filter * table_mask, 1e-9, None))
    tfb = -jnp.log(jnp.clip(1.0 - (1.0 - table_filter) * table_mask, 1e-9, None))
    emn = (table_edge > 0.0).astype(jnp.float32)
    mb = (table_mask - 1.0) * big
    aux = jnp.concatenate([tfa, tfb, table_edge, mb, mb, (emn - 1.0) * big],
                          axis=1)

    grid = n // BLOCK
    out = pl.pallas_call(
        _body,
        grid=(grid,),
        in_specs=[
            pl.BlockSpec((BLOCK, 6 * R), lambda i: (i, 0)),
            pl.BlockSpec((BLOCK, GLOB), lambda i: (i, 0)),
            pl.BlockSpec((BLOCK, ONEHOT), lambda i: (i, 0)),
            pl.BlockSpec((BLOCK, R, C13), lambda i: (i, 0, 0)),
            pl.BlockSpec((ONEHOT, C13 * HID), lambda i: (0, 0)),
            pl.BlockSpec((BLOCK, HID, 3 * NH), lambda i: (0, 0, 0)),
            pl.BlockSpec((BLOCK, 6, 96), lambda i: (0, 0, 0)),
            pl.BlockSpec((3 * NH, FEAT), lambda i: (0, 0)),
            pl.BlockSpec((ONEHOT, FEAT), lambda i: (0, 0)),
            pl.BlockSpec((GLOB, FEAT), lambda i: (0, 0)),
        ],
        out_specs=pl.BlockSpec((BLOCK, FEAT), lambda i: (i, 0)),
        out_shape=jax.ShapeDtypeStruct((n, FEAT), jnp.float32),
    )(aux, table_global, table_onehot,
      table_others, W2, WseB, selB, Wagg, Woh1, Woh2)
    return out


# speedup vs baseline: 6.1623x; 1.0728x over previous
import functools

import jax
import jax.numpy as jnp
from jax.experimental import pallas as pl
from jax.experimental.pallas import tpu as pltpu

N = 8192
R = 65
ONEHOT = 512
GLOB = 32
FEAT = 256
HID = 32
NH = 16
TFS = 64
C13 = 13

BLOCK = 128


def _body(tfil_ref, tmask_ref, tedge_ref, tg_ref, toh_ref, tot_ref,
          W2_ref, Wse_ref, sel_ref, Wagg_ref, Woh1_ref, Woh2_ref, out_ref):
    toh = toh_ref[...]
    M = jnp.dot(toh, W2_ref[...],
                preferred_element_type=jnp.float32)
    B = toh.shape[0]
    M3 = M.reshape(B, HID, C13)
    emb = jax.lax.dot_general(
        tot_ref[...], M3, (((2,), (2,)), ((0,), (0,))),
        preferred_element_type=jnp.float32)
    emb = jnp.maximum(emb, 0.0)
    emb2 = jax.lax.dot_general(
        emb, Wse_ref[...], (((2,), (1,)), ((0,), (0,))),
        preferred_element_type=jnp.float32)

    big = jnp.float32(1e38)
    tfil = tfil_ref[...]
    tmask = tmask_ref[...]
    tedge = tedge_ref[...]
    tfa = -jnp.log(jnp.clip(1.0 - tfil * tmask, 1e-9, None))
    tfb = -jnp.log(jnp.clip(1.0 - (1.0 - tfil) * tmask, 1e-9, None))
    mb = (tmask - 1.0) * big
    eb = ((tedge > 0.0).astype(jnp.float32) - 1.0) * big
    aux3 = jnp.concatenate([tfa, tfb, tedge, mb, mb, eb],
                           axis=1).reshape(B, 6, R)
    sm = jax.lax.dot_general(
        aux3, sel_ref[...], (((1,), (1,)), ((0,), (0,))),
        preferred_element_type=jnp.float32)
    mx = jnp.max(emb2 * sm[:, :, :48] + sm[:, :, 48:], axis=1)
    inv_big = jnp.float32(1e-38)
    m_any = 1.0 + jnp.max(mb, axis=1)[:, None] * inv_big
    e_any = 1.0 + jnp.max(eb, axis=1)[:, None] * inv_big
    anyv = jnp.concatenate([
        jnp.broadcast_to(m_any, (B, 2 * NH)),
        jnp.broadcast_to(e_any, (B, NH))], axis=1)
    agg = mx * anyv
    out = jnp.dot(agg, Wagg_ref[...], preferred_element_type=jnp.float32)
    out = out + jnp.dot(toh, Woh1_ref[...], preferred_element_type=jnp.float32)
    out = out + jnp.dot(tg_ref[...], Woh2_ref[...],
                        preferred_element_type=jnp.float32)
    out_ref[...] = out


@jax.jit
def kernel(table_filter, table_mask, table_edge, table_global, table_onehot,
           table_others, W_onehot_emb, W_schema_prep, W_se, b_se,
           W_onehot_transform, W_edge_fc, W_left, W_right, W_tail):
    n = table_onehot.shape[0]
    W_emb3 = W_onehot_emb.reshape(ONEHOT, HID, HID)
    W2 = jax.lax.dot_general(
        W_emb3, W_schema_prep,
        (((1,), (1,)), ((), ()))).reshape(ONEHOT, HID * C13)
    Wagg = jnp.concatenate([
        W_left @ W_tail[HID:2 * HID],
        W_right @ W_tail[2 * HID:3 * HID],
        W_edge_fc @ W_tail[0:HID]], axis=0)
    Woh = W_onehot_transform @ W_tail[3 * HID:]
    Woh1 = Woh[:ONEHOT]
    Woh2 = Woh[ONEHOT:]
    ID3 = jnp.repeat(jnp.eye(3, dtype=jnp.float32), NH, axis=1)
    Z3 = jnp.zeros((3, 3 * NH), jnp.float32)
    S1 = jnp.concatenate([ID3, Z3], axis=0)
    S2 = jnp.concatenate([Z3, ID3], axis=0)
    sel = jnp.concatenate([S1, S2 + S1 * b_se[None, :]], axis=1)
    selB = jnp.broadcast_to(sel[None], (BLOCK, 6, 96))
    WseB = jnp.broadcast_to(W_se[None], (BLOCK, HID, 3 * NH))

    grid = n // BLOCK
    out = pl.pallas_call(
        _body,
        grid=(grid,),
        in_specs=[
            pl.BlockSpec((BLOCK, R), lambda i: (i, 0)),
            pl.BlockSpec((BLOCK, R), lambda i: (i, 0)),
            pl.BlockSpec((BLOCK, R), lambda i: (i, 0)),
            pl.BlockSpec((BLOCK, GLOB), lambda i: (i, 0)),
            pl.BlockSpec((BLOCK, ONEHOT), lambda i: (i, 0)),
            pl.BlockSpec((BLOCK, R, C13), lambda i: (i, 0, 0)),
            pl.BlockSpec((ONEHOT, C13 * HID), lambda i: (0, 0)),
            pl.BlockSpec((BLOCK, HID, 3 * NH), lambda i: (0, 0, 0)),
            pl.BlockSpec((BLOCK, 6, 96), lambda i: (0, 0, 0)),
            pl.BlockSpec((3 * NH, FEAT), lambda i: (0, 0)),
            pl.BlockSpec((ONEHOT, FEAT), lambda i: (0, 0)),
            pl.BlockSpec((GLOB, FEAT), lambda i: (0, 0)),
        ],
        out_specs=pl.BlockSpec((BLOCK, FEAT), lambda i: (i, 0)),
        out_shape=jax.ShapeDtypeStruct((n, FEAT), jnp.float32),
    )(table_filter, table_mask, table_edge, table_global, table_onehot,
      table_others, W2, WseB, selB, Wagg, Woh1, Woh2)
    return out
